# Initial kernel scaffold; baseline (speedup 1.0000x reference)
import jax
import jax.numpy as jnp
from jax.experimental import pallas as pl


def kernel(int_vec, tables):
    raise NotImplementedError("write your pallas kernel here")


# speedup vs baseline: 1.8787x; 1.8787x over previous
import functools

import jax
import jax.numpy as jnp
from jax import lax
from jax.experimental import pallas as pl
from jax.experimental.pallas import tpu as pltpu, tpu_sc as plsc

INPUT_DIM = 26
NUM_EMB = 100000
OUT_DIM = 32
LANES = 16

ROWS_PER_CHUNK = 64
LOOKUPS = ROWS_PER_CHUNK * INPUT_DIM
IDX_ROWS = LOOKUPS // 128


def _build(num_rows):
    NC, NS = 2, 16
    NW = NC * NS
    rows_per_w = num_rows // NW
    chunks = rows_per_w // ROWS_PER_CHUNK
    idx_rows_per_w = rows_per_w * INPUT_DIM // 128

    mesh = plsc.VectorSubcoreMesh(core_axis_name="c", subcore_axis_name="s")

    @functools.partial(
        pl.kernel,
        mesh=mesh,
        compiler_params=pltpu.CompilerParams(use_tc_tiling_on_sc=False),
        out_type=jax.ShapeDtypeStruct((num_rows, OUT_DIM), jnp.float32),
        scratch_types=[
            pltpu.VMEM((LOOKUPS,), jnp.int32),
            pltpu.VMEM((LOOKUPS,), jnp.int32),
            pltpu.VMEM((LOOKUPS, OUT_DIM), jnp.float32),
            pltpu.VMEM((ROWS_PER_CHUNK, OUT_DIM), jnp.float32),
            pltpu.SemaphoreType.DMA,
        ],
    )
    def k(tab_hbm, idx_hbm, offs_hbm, out_hbm, idx_v, offs_v, rows_v, out_v, sem):
        wid = lax.axis_index("s") * NC + lax.axis_index("c")
        pltpu.sync_copy(offs_hbm, offs_v)

        def chunk_body(g, _):
            off = pl.multiple_of(wid * (rows_per_w * INPUT_DIM) + g * LOOKUPS, 8)
            pltpu.sync_copy(idx_hbm.at[pl.ds(off, LOOKUPS)], idx_v)

            def offs_body(j, _):
                for kk in range(128 // LANES):
                    sl = pl.ds(j * 128 + kk * LANES, LANES)
                    idx_v[sl] = idx_v[sl] + offs_v[sl]
                return 0

            lax.fori_loop(0, IDX_ROWS, offs_body, 0)

            copies = [
                pltpu.async_copy(
                    tab_hbm.at[idx_v.at[pl.ds(j * 128, 128)]],
                    rows_v.at[pl.ds(j * 128, 128)],
                    sem,
                )
                for j in range(IDX_ROWS)
            ]
            for cp in copies:
                cp.wait()

            def acc_body(c, _):
                base = c * INPUT_DIM
                a0 = rows_v[base, pl.ds(0, LANES)]
                a1 = rows_v[base, pl.ds(LANES, LANES)]
                for i in range(1, INPUT_DIM):
                    a0 = a0 + rows_v[base + i, pl.ds(0, LANES)]
                    a1 = a1 + rows_v[base + i, pl.ds(LANES, LANES)]
                out_v[c, pl.ds(0, LANES)] = a0
                out_v[c, pl.ds(LANES, LANES)] = a1
                return 0

            lax.fori_loop(0, ROWS_PER_CHUNK, acc_body, 0)

            pltpu.sync_copy(
                out_v,
                out_hbm.at[pl.ds(wid * rows_per_w + g * ROWS_PER_CHUNK,
                                 ROWS_PER_CHUNK)],
            )
            return 0

        lax.fori_loop(0, chunks, chunk_body, 0)

    return k


def kernel(int_vec, tables):
    bs, num_nodes, input_dim = int_vec.shape
    num_rows = bs * num_nodes
    tab_flat = tables.reshape(input_dim * tables.shape[1], tables.shape[2])
    idx_flat = int_vec.reshape(num_rows * input_dim)
    offs = jnp.tile(
        jnp.arange(INPUT_DIM, dtype=jnp.int32) * NUM_EMB, ROWS_PER_CHUNK
    )
    out = _build(num_rows)(tab_flat, idx_flat, offs)
    return out.reshape(bs, num_nodes, tables.shape[2])


# speedup vs baseline: 1.9607x; 1.0436x over previous
import functools

import jax
import jax.numpy as jnp
from jax import lax
from jax.experimental import pallas as pl
from jax.experimental.pallas import tpu as pltpu, tpu_sc as plsc

INPUT_DIM = 26
NUM_EMB = 100000
OUT_DIM = 32
LANES = 16

ROWS_PER_CHUNK = 64
LOOKUPS = ROWS_PER_CHUNK * INPUT_DIM


def _build(num_rows):
    NC, NS = 2, 16
    NW = NC * NS
    rows_per_w = num_rows // NW
    chunks = rows_per_w // ROWS_PER_CHUNK
    assert chunks % 2 == 1

    mesh = plsc.VectorSubcoreMesh(core_axis_name="c", subcore_axis_name="s")

    @functools.partial(
        pl.kernel,
        mesh=mesh,
        compiler_params=pltpu.CompilerParams(use_tc_tiling_on_sc=False),
        out_type=jax.ShapeDtypeStruct((num_rows, OUT_DIM), jnp.float32),
        scratch_types=[
            pltpu.VMEM((LOOKUPS,), jnp.int32),
            pltpu.VMEM((LOOKUPS,), jnp.int32),
            pltpu.VMEM((LOOKUPS,), jnp.int32),
            pltpu.VMEM((LOOKUPS, OUT_DIM), jnp.float32),
            pltpu.VMEM((LOOKUPS, OUT_DIM), jnp.float32),
            pltpu.VMEM((ROWS_PER_CHUNK, OUT_DIM), jnp.float32),
            pltpu.SemaphoreType.DMA,
            pltpu.SemaphoreType.DMA,
        ],
    )
    def k(tab_hbm, idx_hbm, offs_hbm, out_hbm,
          idx_a, idx_b, offs_v, rows_a, rows_b, out_v, sem_a, sem_b):
        wid = lax.axis_index("s") * NC + lax.axis_index("c")
        pltpu.sync_copy(offs_hbm, offs_v)
        idx_base = wid * (rows_per_w * INPUT_DIM)

        def stage(g, idx_v, rows_v, sem):
            off = pl.multiple_of(idx_base + g * LOOKUPS, 8)
            pltpu.sync_copy(idx_hbm.at[pl.ds(off, LOOKUPS)], idx_v)

            def offs_body(j, _):
                for kk in range(128 // LANES):
                    sl = pl.ds(j * 128 + kk * LANES, LANES)
                    idx_v[sl] = idx_v[sl] + offs_v[sl]
                return 0

            lax.fori_loop(0, LOOKUPS // 128, offs_body, 0)
            return pltpu.async_copy(tab_hbm.at[idx_v], rows_v, sem)

        def process(g, rows_v):
            def acc_body(c, _):
                base = c * INPUT_DIM
                a0 = rows_v[base, pl.ds(0, LANES)]
                a1 = rows_v[base, pl.ds(LANES, LANES)]
                for i in range(1, INPUT_DIM):
                    a0 = a0 + rows_v[base + i, pl.ds(0, LANES)]
                    a1 = a1 + rows_v[base + i, pl.ds(LANES, LANES)]
                out_v[c, pl.ds(0, LANES)] = a0
                out_v[c, pl.ds(LANES, LANES)] = a1
                return 0

            lax.fori_loop(0, ROWS_PER_CHUNK, acc_body, 0)
            pltpu.sync_copy(
                out_v,
                out_hbm.at[pl.ds(wid * rows_per_w + g * ROWS_PER_CHUNK,
                                 ROWS_PER_CHUNK)],
            )

        stage(0, idx_a, rows_a, sem_a)

        def pair_body(t, _):
            g = 2 * t
            cp_b = stage(g + 1, idx_b, rows_b, sem_b)
            pltpu.make_async_copy(tab_hbm.at[idx_a], rows_a, sem_a).wait()
            process(g, rows_a)
            stage(g + 2, idx_a, rows_a, sem_a)
            cp_b.wait()
            process(g + 1, rows_b)
            return 0

        lax.fori_loop(0, chunks // 2, pair_body, 0)
        pltpu.make_async_copy(tab_hbm.at[idx_a], rows_a, sem_a).wait()
        process(chunks - 1, rows_a)

    return k


def kernel(int_vec, tables):
    bs, num_nodes, input_dim = int_vec.shape
    num_rows = bs * num_nodes
    tab_flat = tables.reshape(input_dim * tables.shape[1], tables.shape[2])
    idx_flat = int_vec.reshape(num_rows * input_dim)
    offs = jnp.tile(
        jnp.arange(INPUT_DIM, dtype=jnp.int32) * NUM_EMB, ROWS_PER_CHUNK
    )
    out = _build(num_rows)(tab_flat, idx_flat, offs)
    return out.reshape(bs, num_nodes, tables.shape[2])
